# baseline (device time: 76572 ns/iter reference)
import jax
import jax.numpy as jnp
from jax import lax
from jax.experimental import pallas as pl
from jax.experimental.pallas import tpu as pltpu

N_DEV = 4
SQ = 256
SKV = 4096
H_LOC = 8
DH = 128
QB = 64
N_QB = SQ // QB
KV_PER_QB = SKV // N_QB
D_MODEL = 1024
SCALE = 0.08838834764831843


def kernel(x, Wq, K_ext, V_ext, Wo):
    my = lax.axis_index("i")

    x2d = x.reshape(SQ, D_MODEL).astype(jnp.bfloat16)
    wq_loc = lax.dynamic_slice(
        Wq, (0, my * H_LOC * DH), (D_MODEL, H_LOC * DH)
    ).astype(jnp.bfloat16)
    wo_b = Wo.astype(jnp.bfloat16)

    def gather(a):
        a = a.reshape(SKV // (4 * QB), 4, QB, H_LOC, DH)
        a = a.transpose(1, 3, 0, 2, 4)
        return a.reshape(N_QB, H_LOC, KV_PER_QB, DH).astype(jnp.bfloat16)

    k_g = gather(K_ext)
    v_g = gather(V_ext)

    def body(x_ref, wq_ref, k_ref, v_ref, wo_ref, out_ref,
             comm_ref, send_sems, recv_sems):
        my_pos = lax.axis_index("i")
        left = lax.rem(my_pos - 1 + N_DEV, N_DEV)
        right = lax.rem(my_pos + 1, N_DEV)

        barrier_sem = pltpu.get_barrier_semaphore()
        for nbr in (left, right):
            pl.semaphore_signal(
                barrier_sem, inc=1,
                device_id=(nbr,), device_id_type=pl.DeviceIdType.MESH,
            )
        pl.semaphore_wait(barrier_sem, 2)

        q = jnp.dot(x_ref[...], wq_ref[...],
                    preferred_element_type=jnp.float32)

        ctx_rows = []
        for qb in range(N_QB):
            head_ctx = []
            for h in range(H_LOC):
                q_h = q[qb * QB:(qb + 1) * QB,
                        h * DH:(h + 1) * DH].astype(jnp.bfloat16)
                k_h = k_ref[qb, h]
                s = lax.dot_general(
                    q_h, k_h, (((1,), (1,)), ((), ())),
                    preferred_element_type=jnp.float32,
                ) * SCALE
                m = jnp.max(s, axis=1, keepdims=True)
                w = jnp.exp(s - m)
                w = w / jnp.sum(w, axis=1, keepdims=True)
                ctx_h = jnp.dot(w.astype(jnp.bfloat16), v_ref[qb, h],
                                preferred_element_type=jnp.float32)
                head_ctx.append(ctx_h)
            ctx_rows.append(jnp.concatenate(head_ctx, axis=1))
        ctx = jnp.concatenate(ctx_rows, axis=0).astype(jnp.bfloat16)

        comm_ref[0] = ctx
        out_ref[...] = jnp.dot(
            ctx, wo_ref[pl.ds(my_pos * D_MODEL, D_MODEL), :],
            preferred_element_type=jnp.float32,
        )

        for hop in range(N_DEV - 1):
            send_slot = hop % 2
            recv_slot = (hop + 1) % 2
            rdma = pltpu.make_async_remote_copy(
                src_ref=comm_ref.at[send_slot],
                dst_ref=comm_ref.at[recv_slot],
                send_sem=send_sems.at[send_slot],
                recv_sem=recv_sems.at[recv_slot],
                device_id=(right,),
                device_id_type=pl.DeviceIdType.MESH,
            )
            rdma.start()
            rdma.wait()

            origin = lax.rem(my_pos - hop - 1 + N_DEV, N_DEV)
            out_ref[...] += jnp.dot(
                comm_ref[recv_slot],
                wo_ref[pl.ds(origin * D_MODEL, D_MODEL), :],
                preferred_element_type=jnp.float32,
            )

    out2d = pl.pallas_call(
        body,
        out_shape=jax.ShapeDtypeStruct((SQ, D_MODEL), jnp.float32),
        in_specs=[pl.BlockSpec(memory_space=pltpu.VMEM)] * 5,
        out_specs=pl.BlockSpec(memory_space=pltpu.VMEM),
        scratch_shapes=[
            pltpu.VMEM((2, SQ, D_MODEL), jnp.bfloat16),
            pltpu.SemaphoreType.DMA((2,)),
            pltpu.SemaphoreType.DMA((2,)),
        ],
        compiler_params=pltpu.CompilerParams(collective_id=0),
    )(x2d, wq_loc, k_g, v_g, wo_b)

    return out2d.reshape(1, SQ, D_MODEL)


# device time: 65875 ns/iter; 1.1624x vs baseline; 1.1624x over previous
import jax
import jax.numpy as jnp
from jax import lax
from jax.experimental import pallas as pl
from jax.experimental.pallas import tpu as pltpu

N_DEV = 4
SQ = 256
SKV = 4096
H_LOC = 8
DH = 128
QB = 64
N_QB = SQ // QB
KV_PER_QB = SKV // N_QB
D_MODEL = 1024
SCALE = 0.08838834764831843


def kernel(x, Wq, K_ext, V_ext, Wo):
    my = lax.axis_index("i")

    x2d = x.reshape(SQ, D_MODEL).astype(jnp.bfloat16)
    wq_loc = lax.dynamic_slice(
        Wq, (0, my * H_LOC * DH), (D_MODEL, H_LOC * DH)
    ).astype(jnp.bfloat16)
    wo_b = Wo.astype(jnp.bfloat16)

    def gather(a):
        a = a.reshape(SKV // (4 * QB), 4, QB, H_LOC, DH)
        a = a.transpose(1, 3, 0, 2, 4)
        return a.reshape(N_QB, H_LOC, KV_PER_QB, DH).astype(jnp.bfloat16)

    k_g = gather(K_ext)
    v_g = gather(V_ext)

    def body(x_ref, wq_ref, k_ref, v_ref, wo_ref, out_ref,
             comm_r, comm_l, send_r, recv_r, send_l, recv_l):
        my_pos = lax.axis_index("i")
        left = lax.rem(my_pos - 1 + N_DEV, N_DEV)
        right = lax.rem(my_pos + 1, N_DEV)

        barrier_sem = pltpu.get_barrier_semaphore()
        for nbr in (left, right):
            pl.semaphore_signal(
                barrier_sem, inc=1,
                device_id=(nbr,), device_id_type=pl.DeviceIdType.MESH,
            )
        pl.semaphore_wait(barrier_sem, 2)

        q = jnp.dot(x_ref[...], wq_ref[...],
                    preferred_element_type=jnp.float32)

        ctx_rows = []
        for qb in range(N_QB):
            head_ctx = []
            for h in range(H_LOC):
                q_h = q[qb * QB:(qb + 1) * QB,
                        h * DH:(h + 1) * DH].astype(jnp.bfloat16)
                k_h = k_ref[qb, h]
                s = lax.dot_general(
                    q_h, k_h, (((1,), (1,)), ((), ())),
                    preferred_element_type=jnp.float32,
                ) * SCALE
                m = jnp.max(s, axis=1, keepdims=True)
                w = jnp.exp(s - m)
                w = w / jnp.sum(w, axis=1, keepdims=True)
                ctx_h = jnp.dot(w.astype(jnp.bfloat16), v_ref[qb, h],
                                preferred_element_type=jnp.float32)
                head_ctx.append(ctx_h)
            ctx_rows.append(jnp.concatenate(head_ctx, axis=1))
        ctx = jnp.concatenate(ctx_rows, axis=0).astype(jnp.bfloat16)

        half = D_MODEL // 2
        comm_r[0] = ctx[:, :half]
        comm_l[0] = ctx[:, half:]

        def make(hop, comm, sems_s, sems_r, dst):
            return pltpu.make_async_remote_copy(
                src_ref=comm.at[hop],
                dst_ref=comm.at[hop + 1],
                send_sem=sems_s.at[hop],
                recv_sem=sems_r.at[hop],
                device_id=(dst,),
                device_id_type=pl.DeviceIdType.MESH,
            )

        rd_r = [make(h, comm_r, send_r, recv_r, right) for h in range(3)]
        rd_l = [make(h, comm_l, send_l, recv_l, left) for h in range(3)]
        rd_r[0].start()
        rd_l[0].start()

        out_ref[...] = jnp.dot(
            ctx, wo_ref[pl.ds(my_pos * D_MODEL, D_MODEL), :],
            preferred_element_type=jnp.float32,
        )

        for hop in range(3):
            rd_r[hop].wait_recv()
            if hop < 2:
                rd_r[hop + 1].start()
            org_r = lax.rem(my_pos - hop - 1 + N_DEV, N_DEV)
            out_ref[...] += jnp.dot(
                comm_r[hop + 1],
                wo_ref[pl.ds(org_r * D_MODEL, half), :],
                preferred_element_type=jnp.float32,
            )
            rd_l[hop].wait_recv()
            if hop < 2:
                rd_l[hop + 1].start()
            org_l = lax.rem(my_pos + hop + 1, N_DEV)
            out_ref[...] += jnp.dot(
                comm_l[hop + 1],
                wo_ref[pl.ds(org_l * D_MODEL + half, half), :],
                preferred_element_type=jnp.float32,
            )

        for hop in range(3):
            rd_r[hop].wait_send()
            rd_l[hop].wait_send()

    out2d = pl.pallas_call(
        body,
        out_shape=jax.ShapeDtypeStruct((SQ, D_MODEL), jnp.float32),
        in_specs=[pl.BlockSpec(memory_space=pltpu.VMEM)] * 5,
        out_specs=pl.BlockSpec(memory_space=pltpu.VMEM),
        scratch_shapes=[
            pltpu.VMEM((4, SQ, D_MODEL // 2), jnp.bfloat16),
            pltpu.VMEM((4, SQ, D_MODEL // 2), jnp.bfloat16),
            pltpu.SemaphoreType.DMA((3,)),
            pltpu.SemaphoreType.DMA((3,)),
            pltpu.SemaphoreType.DMA((3,)),
            pltpu.SemaphoreType.DMA((3,)),
        ],
        compiler_params=pltpu.CompilerParams(collective_id=0),
    )(x2d, wq_loc, k_g, v_g, wo_b)

    return out2d.reshape(1, SQ, D_MODEL)


# device time: 65737 ns/iter; 1.1648x vs baseline; 1.0021x over previous
import jax
import jax.numpy as jnp
from jax import lax
from jax.experimental import pallas as pl
from jax.experimental.pallas import tpu as pltpu

N_DEV = 4
SQ = 256
SKV = 4096
H_LOC = 8
DH = 128
QB = 64
N_QB = SQ // QB
KV_PER_QB = SKV // N_QB
D_MODEL = 1024
SCALE = 0.08838834764831843


def kernel(x, Wq, K_ext, V_ext, Wo):
    my = lax.axis_index("i")

    x2d = x.reshape(SQ, D_MODEL).astype(jnp.bfloat16)
    wq_loc = lax.dynamic_slice(
        Wq, (0, my * H_LOC * DH), (D_MODEL, H_LOC * DH)
    ).astype(jnp.bfloat16)
    k_b = K_ext.reshape(SKV, H_LOC * DH).astype(jnp.bfloat16)
    v_b = V_ext.reshape(SKV, H_LOC * DH).astype(jnp.bfloat16)

    def body(x_ref, wq_ref, k_ref, v_ref, wo_ref, out_ref,
             comm_r, comm_l, send_r, recv_r, send_l, recv_l):
        my_pos = lax.axis_index("i")
        left = lax.rem(my_pos - 1 + N_DEV, N_DEV)
        right = lax.rem(my_pos + 1, N_DEV)

        barrier_sem = pltpu.get_barrier_semaphore()
        for nbr in (left, right):
            pl.semaphore_signal(
                barrier_sem, inc=1,
                device_id=(nbr,), device_id_type=pl.DeviceIdType.MESH,
            )
        pl.semaphore_wait(barrier_sem, 2)

        q = jnp.dot(x_ref[...], wq_ref[...],
                    preferred_element_type=jnp.float32)

        ctx_rows = []
        for qb in range(N_QB):
            k_qb = jnp.concatenate(
                [k_ref[(4 * t + qb) * QB:(4 * t + qb + 1) * QB, :]
                 for t in range(KV_PER_QB // QB)], axis=0)
            v_qb = jnp.concatenate(
                [v_ref[(4 * t + qb) * QB:(4 * t + qb + 1) * QB, :]
                 for t in range(KV_PER_QB // QB)], axis=0)
            head_ctx = []
            for h in range(H_LOC):
                q_h = q[qb * QB:(qb + 1) * QB,
                        h * DH:(h + 1) * DH].astype(jnp.bfloat16)
                k_h = k_qb[:, h * DH:(h + 1) * DH]
                s = lax.dot_general(
                    q_h, k_h, (((1,), (1,)), ((), ())),
                    preferred_element_type=jnp.float32,
                ) * SCALE
                m = jnp.max(s, axis=1, keepdims=True)
                w = jnp.exp(s - m)
                w = w / jnp.sum(w, axis=1, keepdims=True)
                ctx_h = jnp.dot(w.astype(jnp.bfloat16),
                                v_qb[:, h * DH:(h + 1) * DH],
                                preferred_element_type=jnp.float32)
                head_ctx.append(ctx_h)
            ctx_rows.append(jnp.concatenate(head_ctx, axis=1))
        ctx = jnp.concatenate(ctx_rows, axis=0).astype(jnp.bfloat16)

        half = D_MODEL // 2
        comm_r[0] = ctx[:, :half]
        comm_l[0] = ctx[:, half:]

        def make(hop, comm, sems_s, sems_r, dst):
            return pltpu.make_async_remote_copy(
                src_ref=comm.at[hop],
                dst_ref=comm.at[hop + 1],
                send_sem=sems_s.at[hop],
                recv_sem=sems_r.at[hop],
                device_id=(dst,),
                device_id_type=pl.DeviceIdType.MESH,
            )

        rd_r = [make(h, comm_r, send_r, recv_r, right) for h in range(3)]
        rd_l = [make(h, comm_l, send_l, recv_l, left) for h in range(3)]
        rd_r[0].start()
        rd_l[0].start()

        out_ref[...] = jnp.dot(
            ctx,
            wo_ref[pl.ds(my_pos * D_MODEL, D_MODEL), :].astype(jnp.bfloat16),
            preferred_element_type=jnp.float32,
        )

        for hop in range(3):
            rd_r[hop].wait_recv()
            if hop < 2:
                rd_r[hop + 1].start()
            org_r = lax.rem(my_pos - hop - 1 + N_DEV, N_DEV)
            out_ref[...] += jnp.dot(
                comm_r[hop + 1],
                wo_ref[pl.ds(org_r * D_MODEL, half), :].astype(jnp.bfloat16),
                preferred_element_type=jnp.float32,
            )
            rd_l[hop].wait_recv()
            if hop < 2:
                rd_l[hop + 1].start()
            org_l = lax.rem(my_pos + hop + 1, N_DEV)
            out_ref[...] += jnp.dot(
                comm_l[hop + 1],
                wo_ref[pl.ds(org_l * D_MODEL + half, half), :].astype(
                    jnp.bfloat16),
                preferred_element_type=jnp.float32,
            )

        for hop in range(3):
            rd_r[hop].wait_send()
            rd_l[hop].wait_send()

    out2d = pl.pallas_call(
        body,
        out_shape=jax.ShapeDtypeStruct((SQ, D_MODEL), jnp.float32),
        in_specs=[pl.BlockSpec(memory_space=pltpu.VMEM)] * 5,
        out_specs=pl.BlockSpec(memory_space=pltpu.VMEM),
        scratch_shapes=[
            pltpu.VMEM((4, SQ, D_MODEL // 2), jnp.bfloat16),
            pltpu.VMEM((4, SQ, D_MODEL // 2), jnp.bfloat16),
            pltpu.SemaphoreType.DMA((3,)),
            pltpu.SemaphoreType.DMA((3,)),
            pltpu.SemaphoreType.DMA((3,)),
            pltpu.SemaphoreType.DMA((3,)),
        ],
        compiler_params=pltpu.CompilerParams(collective_id=0),
    )(x2d, wq_loc, k_b, v_b, Wo)

    return out2d.reshape(1, SQ, D_MODEL)


# device time: 48327 ns/iter; 1.5845x vs baseline; 1.3603x over previous
import jax
import jax.numpy as jnp
from jax import lax
from jax.experimental import pallas as pl
from jax.experimental.pallas import tpu as pltpu

N_DEV = 4
SQ = 256
SKV = 4096
H_LOC = 8
DH = 128
QB = 64
N_QB = SQ // QB
KV_PER_QB = SKV // N_QB
NT = KV_PER_QB // QB
D_MODEL = 1024
SCALE = 0.08838834764831843


def kernel(x, Wq, K_ext, V_ext, Wo):
    x2d = x.reshape(SQ, D_MODEL)
    k5 = K_ext.reshape(NT, N_QB, QB, H_LOC, DH)
    v5 = V_ext.reshape(NT, N_QB, QB, H_LOC, DH)

    def body(x_ref, wq_hbm, k_hbm, v_hbm, wo_hbm, out_ref,
             wq_scr, k_scr, v_scr, wo_scr,
             wq_sem, wo_sem, k_sems, v_sems,
             comm_r, comm_l, send_r, recv_r, send_l, recv_l):
        my_pos = lax.axis_index("i")
        left = lax.rem(my_pos - 1 + N_DEV, N_DEV)
        right = lax.rem(my_pos + 1, N_DEV)

        barrier_sem = pltpu.get_barrier_semaphore()
        for nbr in (left, right):
            pl.semaphore_signal(
                barrier_sem, inc=1,
                device_id=(nbr,), device_id_type=pl.DeviceIdType.MESH,
            )
        pl.semaphore_wait(barrier_sem, 2)

        wq_dma = pltpu.make_async_copy(
            wq_hbm.at[:, pl.ds(my_pos * H_LOC * DH, H_LOC * DH)],
            wq_scr, wq_sem)
        wq_dma.start()
        wo_dma = pltpu.make_async_copy(wo_hbm, wo_scr, wo_sem)
        wo_dma.start()

        def issue_kv(qb, slot):
            dmas = []
            for h in range(H_LOC):
                dmas.append(pltpu.make_async_copy(
                    k_hbm.at[:, qb, :, h, :], k_scr.at[slot, h],
                    k_sems.at[slot]))
                dmas.append(pltpu.make_async_copy(
                    v_hbm.at[:, qb, :, h, :], v_scr.at[slot, h],
                    v_sems.at[slot]))
            for d in dmas:
                d.start()
            return dmas

        kv = {0: issue_kv(0, 0), 1: issue_kv(1, 1)}

        wq_dma.wait()
        q = jnp.dot(x_ref[...].astype(jnp.bfloat16),
                    wq_scr[...].astype(jnp.bfloat16),
                    preferred_element_type=jnp.float32)

        ctx_rows = []
        for qb in range(N_QB):
            slot = qb % 2
            for d in kv[qb]:
                d.wait()
            k_qb = k_scr[slot].astype(jnp.bfloat16).reshape(
                H_LOC, KV_PER_QB, DH)
            v_qb = v_scr[slot].astype(jnp.bfloat16).reshape(
                H_LOC, KV_PER_QB, DH)
            head_ctx = []
            for h in range(H_LOC):
                q_h = q[qb * QB:(qb + 1) * QB,
                        h * DH:(h + 1) * DH].astype(jnp.bfloat16)
                s = lax.dot_general(
                    q_h, k_qb[h], (((1,), (1,)), ((), ())),
                    preferred_element_type=jnp.float32,
                ) * SCALE
                m = jnp.max(s, axis=1, keepdims=True)
                w = jnp.exp(s - m)
                w = w / jnp.sum(w, axis=1, keepdims=True)
                ctx_h = jnp.dot(w.astype(jnp.bfloat16), v_qb[h],
                                preferred_element_type=jnp.float32)
                head_ctx.append(ctx_h)
            if qb + 2 < N_QB:
                kv[qb + 2] = issue_kv(qb + 2, slot)
            ctx_rows.append(jnp.concatenate(head_ctx, axis=1))
        ctx = jnp.concatenate(ctx_rows, axis=0).astype(jnp.bfloat16)

        half = D_MODEL // 2
        comm_r[0] = ctx[:, :half]
        comm_l[0] = ctx[:, half:]

        def make(hop, comm, sems_s, sems_r, dst):
            return pltpu.make_async_remote_copy(
                src_ref=comm.at[hop],
                dst_ref=comm.at[hop + 1],
                send_sem=sems_s.at[hop],
                recv_sem=sems_r.at[hop],
                device_id=(dst,),
                device_id_type=pl.DeviceIdType.MESH,
            )

        rd_r = [make(h, comm_r, send_r, recv_r, right) for h in range(3)]
        rd_l = [make(h, comm_l, send_l, recv_l, left) for h in range(3)]
        rd_r[0].start()
        rd_l[0].start()

        wo_dma.wait()
        out_ref[...] = jnp.dot(
            ctx,
            wo_scr[pl.ds(my_pos * D_MODEL, D_MODEL), :].astype(jnp.bfloat16),
            preferred_element_type=jnp.float32,
        )

        for hop in range(3):
            rd_r[hop].wait_recv()
            if hop < 2:
                rd_r[hop + 1].start()
            org_r = lax.rem(my_pos - hop - 1 + N_DEV, N_DEV)
            out_ref[...] += jnp.dot(
                comm_r[hop + 1],
                wo_scr[pl.ds(org_r * D_MODEL, half), :].astype(jnp.bfloat16),
                preferred_element_type=jnp.float32,
            )
            rd_l[hop].wait_recv()
            if hop < 2:
                rd_l[hop + 1].start()
            org_l = lax.rem(my_pos + hop + 1, N_DEV)
            out_ref[...] += jnp.dot(
                comm_l[hop + 1],
                wo_scr[pl.ds(org_l * D_MODEL + half, half), :].astype(
                    jnp.bfloat16),
                preferred_element_type=jnp.float32,
            )

        for hop in range(3):
            rd_r[hop].wait_send()
            rd_l[hop].wait_send()

    out2d = pl.pallas_call(
        body,
        out_shape=jax.ShapeDtypeStruct((SQ, D_MODEL), jnp.float32),
        in_specs=[
            pl.BlockSpec(memory_space=pltpu.VMEM),
            pl.BlockSpec(memory_space=pl.ANY),
            pl.BlockSpec(memory_space=pl.ANY),
            pl.BlockSpec(memory_space=pl.ANY),
            pl.BlockSpec(memory_space=pl.ANY),
        ],
        out_specs=pl.BlockSpec(memory_space=pltpu.VMEM),
        scratch_shapes=[
            pltpu.VMEM((D_MODEL, H_LOC * DH), jnp.float32),
            pltpu.VMEM((2, H_LOC, NT, QB, DH), jnp.float32),
            pltpu.VMEM((2, H_LOC, NT, QB, DH), jnp.float32),
            pltpu.VMEM((SKV, D_MODEL), jnp.float32),
            pltpu.SemaphoreType.DMA,
            pltpu.SemaphoreType.DMA,
            pltpu.SemaphoreType.DMA((2,)),
            pltpu.SemaphoreType.DMA((2,)),
            pltpu.VMEM((4, SQ, D_MODEL // 2), jnp.bfloat16),
            pltpu.VMEM((4, SQ, D_MODEL // 2), jnp.bfloat16),
            pltpu.SemaphoreType.DMA((3,)),
            pltpu.SemaphoreType.DMA((3,)),
            pltpu.SemaphoreType.DMA((3,)),
            pltpu.SemaphoreType.DMA((3,)),
        ],
        compiler_params=pltpu.CompilerParams(
            collective_id=0, vmem_limit_bytes=60 * 1024 * 1024),
    )(x2d, Wq, k5, v5, Wo)

    return out2d.reshape(1, SQ, D_MODEL)


# device time: 37015 ns/iter; 2.0687x vs baseline; 1.3056x over previous
import jax
import jax.numpy as jnp
from jax import lax
from jax.experimental import pallas as pl
from jax.experimental.pallas import tpu as pltpu

N_DEV = 4
SQ = 256
SKV = 4096
H_LOC = 8
DH = 128
QB = 64
N_QB = SQ // QB
KV_PER_QB = SKV // N_QB
NT = KV_PER_QB // QB
D_MODEL = 1024
QCOL = D_MODEL // 4
SCALE = 0.08838834764831843


def kernel(x, Wq, K_ext, V_ext, Wo):
    x2d = x.reshape(SQ, D_MODEL)
    k5 = K_ext.reshape(NT, N_QB, QB, H_LOC, DH)
    v5 = V_ext.reshape(NT, N_QB, QB, H_LOC, DH)

    def body(x_ref, wq_hbm, k_hbm, v_hbm, wo_hbm, out_ref,
             wq_scr, k_scr, v_scr, wo_scr,
             wq_sem, wo_sem, k_sems, v_sems,
             comm, send_sems, recv_sems):
        my_pos = lax.axis_index("i")
        left = lax.rem(my_pos - 1 + N_DEV, N_DEV)
        right = lax.rem(my_pos + 1, N_DEV)

        barrier_sem = pltpu.get_barrier_semaphore()
        for nbr in (left, right):
            pl.semaphore_signal(
                barrier_sem, inc=1,
                device_id=(nbr,), device_id_type=pl.DeviceIdType.MESH,
            )
        pl.semaphore_wait(barrier_sem, 2)

        wq_dma = pltpu.make_async_copy(
            wq_hbm.at[:, pl.ds(my_pos * H_LOC * DH, H_LOC * DH)],
            wq_scr, wq_sem)
        wq_dma.start()

        def issue_unit(u, slot):
            quarter, qb = divmod(u, N_QB)
            dmas = []
            for i in range(2):
                h = 2 * quarter + i
                dmas.append(pltpu.make_async_copy(
                    k_hbm.at[:, qb, :, h, :], k_scr.at[slot, i],
                    k_sems.at[slot]))
                dmas.append(pltpu.make_async_copy(
                    v_hbm.at[:, qb, :, h, :], v_scr.at[slot, i],
                    v_sems.at[slot]))
            for d in dmas:
                d.start()
            return dmas

        kv = {u: issue_unit(u, u % 4) for u in range(4)}
        wo_dma = pltpu.make_async_copy(wo_hbm, wo_scr, wo_sem)
        wo_dma.start()

        def mk(ring, hop):
            dst = right if ring % 2 == 0 else left
            return pltpu.make_async_remote_copy(
                src_ref=comm.at[ring, hop],
                dst_ref=comm.at[ring, hop + 1],
                send_sem=send_sems.at[ring, hop],
                recv_sem=recv_sems.at[ring, hop],
                device_id=(dst,),
                device_id_type=pl.DeviceIdType.MESH,
            )

        rd = [[mk(r, h) for h in range(3)] for r in range(4)]

        wq_dma.wait()
        q16 = (jnp.dot(x_ref[...].astype(jnp.bfloat16),
                       wq_scr[...].astype(jnp.bfloat16),
                       preferred_element_type=jnp.float32)
               * SCALE).astype(jnp.bfloat16)

        for quarter in range(4):
            rows = []
            for qb in range(N_QB):
                u = quarter * N_QB + qb
                slot = u % 4
                for d in kv[u]:
                    d.wait()
                k_u = k_scr[slot].astype(jnp.bfloat16).reshape(
                    2, KV_PER_QB, DH)
                v_u = v_scr[slot].astype(jnp.bfloat16).reshape(
                    2, KV_PER_QB, DH)
                pair_ctx = []
                for i in range(2):
                    h = 2 * quarter + i
                    q_h = q16[qb * QB:(qb + 1) * QB, h * DH:(h + 1) * DH]
                    s = lax.dot_general(
                        q_h, k_u[i], (((1,), (1,)), ((), ())),
                        preferred_element_type=jnp.float32)
                    w = jnp.exp(s)
                    denom = jnp.sum(w, axis=1, keepdims=True)
                    ctx_h = jnp.dot(w.astype(jnp.bfloat16), v_u[i],
                                    preferred_element_type=jnp.float32
                                    ) / denom
                    pair_ctx.append(ctx_h)
                if u + 4 < 16:
                    kv[u + 4] = issue_unit(u + 4, slot)
                rows.append(jnp.concatenate(pair_ctx, axis=1))
            ctx_q = jnp.concatenate(rows, axis=0).astype(jnp.bfloat16)
            comm[quarter, 0] = ctx_q
            rd[quarter][0].start()

            if quarter == 0:
                wo_dma.wait()
            contrib = jnp.dot(
                ctx_q,
                wo_scr[pl.ds(my_pos * D_MODEL + quarter * QCOL, QCOL),
                       :].astype(jnp.bfloat16),
                preferred_element_type=jnp.float32,
            )
            if quarter == 0:
                out_ref[...] = contrib
            else:
                out_ref[...] += contrib

        for hop in range(3):
            for ring in range(4):
                rd[ring][hop].wait_recv()
                if hop < 2:
                    rd[ring][hop + 1].start()
                if ring % 2 == 0:
                    org = lax.rem(my_pos - hop - 1 + N_DEV, N_DEV)
                else:
                    org = lax.rem(my_pos + hop + 1, N_DEV)
                out_ref[...] += jnp.dot(
                    comm[ring, hop + 1],
                    wo_scr[pl.ds(org * D_MODEL + ring * QCOL, QCOL),
                           :].astype(jnp.bfloat16),
                    preferred_element_type=jnp.float32,
                )

        for hop in range(3):
            for ring in range(4):
                rd[ring][hop].wait_send()

    out2d = pl.pallas_call(
        body,
        out_shape=jax.ShapeDtypeStruct((SQ, D_MODEL), jnp.float32),
        in_specs=[
            pl.BlockSpec(memory_space=pltpu.VMEM),
            pl.BlockSpec(memory_space=pl.ANY),
            pl.BlockSpec(memory_space=pl.ANY),
            pl.BlockSpec(memory_space=pl.ANY),
            pl.BlockSpec(memory_space=pl.ANY),
        ],
        out_specs=pl.BlockSpec(memory_space=pltpu.VMEM),
        scratch_shapes=[
            pltpu.VMEM((D_MODEL, H_LOC * DH), jnp.float32),
            pltpu.VMEM((4, 2, NT, QB, DH), jnp.float32),
            pltpu.VMEM((4, 2, NT, QB, DH), jnp.float32),
            pltpu.VMEM((SKV, D_MODEL), jnp.float32),
            pltpu.SemaphoreType.DMA,
            pltpu.SemaphoreType.DMA,
            pltpu.SemaphoreType.DMA((4,)),
            pltpu.SemaphoreType.DMA((4,)),
            pltpu.VMEM((4, 4, SQ, QCOL), jnp.bfloat16),
            pltpu.SemaphoreType.DMA((4, 3)),
            pltpu.SemaphoreType.DMA((4, 3)),
        ],
        compiler_params=pltpu.CompilerParams(
            collective_id=0, vmem_limit_bytes=60 * 1024 * 1024),
    )(x2d, Wq, k5, v5, Wo)

    return out2d.reshape(1, SQ, D_MODEL)


# device time: 34508 ns/iter; 2.2190x vs baseline; 1.0726x over previous
import jax
import jax.numpy as jnp
from jax import lax
from jax.experimental import pallas as pl
from jax.experimental.pallas import tpu as pltpu

N_DEV = 4
SQ = 256
SKV = 4096
H_LOC = 8
DH = 128
QB = 64
N_QB = SQ // QB
KV_PER_QB = SKV // N_QB
NT = KV_PER_QB // QB
D_MODEL = 1024
QCOL = D_MODEL // 4
SCALE = 0.08838834764831843


def kernel(x, Wq, K_ext, V_ext, Wo):
    x2d = x.reshape(SQ, D_MODEL)
    k5 = K_ext.reshape(NT, N_QB, QB, H_LOC, DH)
    v5 = V_ext.reshape(NT, N_QB, QB, H_LOC, DH)

    def body(x_ref, wq_hbm, k_hbm, v_hbm, wo_hbm, out_ref,
             wq_scr, k_scr, v_scr, wo_scr,
             wq_sem, wo_sem, k_sems, v_sems,
             snd, rcv, send_sems, recv_sems):
        my_pos = lax.axis_index("i")

        barrier_sem = pltpu.get_barrier_semaphore()
        for o in range(1, N_DEV):
            pl.semaphore_signal(
                barrier_sem, inc=1,
                device_id=(lax.rem(my_pos + o, N_DEV),),
                device_id_type=pl.DeviceIdType.MESH,
            )
        pl.semaphore_wait(barrier_sem, N_DEV - 1)

        wq_dma = pltpu.make_async_copy(
            wq_hbm.at[:, pl.ds(my_pos * H_LOC * DH, H_LOC * DH)],
            wq_scr, wq_sem)
        wq_dma.start()

        def issue_unit(u, slot):
            quarter, qb = divmod(u, N_QB)
            dmas = []
            for i in range(2):
                h = 2 * quarter + i
                dmas.append(pltpu.make_async_copy(
                    k_hbm.at[:, qb, :, h, :], k_scr.at[slot, i],
                    k_sems.at[slot]))
                dmas.append(pltpu.make_async_copy(
                    v_hbm.at[:, qb, :, h, :], v_scr.at[slot, i],
                    v_sems.at[slot]))
            for d in dmas:
                d.start()
            return dmas

        kv = {u: issue_unit(u, u % 4) for u in range(4)}
        wo_dma = pltpu.make_async_copy(wo_hbm, wo_scr, wo_sem)
        wo_dma.start()

        def mk_send(q, o):
            return pltpu.make_async_remote_copy(
                src_ref=snd.at[q],
                dst_ref=rcv.at[q, o - 1],
                send_sem=send_sems.at[q, o - 1],
                recv_sem=recv_sems.at[q, o - 1],
                device_id=(lax.rem(my_pos + o, N_DEV),),
                device_id_type=pl.DeviceIdType.MESH,
            )

        sends = [[mk_send(q, o) for o in range(1, N_DEV)] for q in range(4)]

        def drain(q):
            for s in range(N_DEV - 1):
                sends[q][s].wait_recv()
                org = lax.rem(my_pos - (s + 1) + N_DEV, N_DEV)
                out_ref[...] += jnp.dot(
                    rcv[q, s],
                    wo_scr[pl.ds(org * D_MODEL + q * QCOL, QCOL),
                           :].astype(jnp.bfloat16),
                    preferred_element_type=jnp.float32,
                )

        wq_dma.wait()
        q16 = (jnp.dot(x_ref[...].astype(jnp.bfloat16),
                       wq_scr[...].astype(jnp.bfloat16),
                       preferred_element_type=jnp.float32)
               * SCALE).astype(jnp.bfloat16)

        for quarter in range(4):
            rows = []
            for qb in range(N_QB):
                u = quarter * N_QB + qb
                slot = u % 4
                for d in kv[u]:
                    d.wait()
                k_u = k_scr[slot].astype(jnp.bfloat16).reshape(
                    2, KV_PER_QB, DH)
                v_u = v_scr[slot].astype(jnp.bfloat16).reshape(
                    2, KV_PER_QB, DH)
                pair_ctx = []
                for i in range(2):
                    h = 2 * quarter + i
                    q_h = q16[qb * QB:(qb + 1) * QB, h * DH:(h + 1) * DH]
                    s = lax.dot_general(
                        q_h, k_u[i], (((1,), (1,)), ((), ())),
                        preferred_element_type=jnp.float32)
                    w = jnp.exp(s)
                    denom = jnp.sum(w, axis=1, keepdims=True)
                    ctx_h = jnp.dot(w.astype(jnp.bfloat16), v_u[i],
                                    preferred_element_type=jnp.float32
                                    ) / denom
                    pair_ctx.append(ctx_h)
                if u + 4 < 16:
                    kv[u + 4] = issue_unit(u + 4, slot)
                rows.append(jnp.concatenate(pair_ctx, axis=1))
            ctx_q = jnp.concatenate(rows, axis=0).astype(jnp.bfloat16)
            snd[quarter] = ctx_q
            for s in sends[quarter]:
                s.start()

            if quarter == 0:
                wo_dma.wait()
            contrib = jnp.dot(
                ctx_q,
                wo_scr[pl.ds(my_pos * D_MODEL + quarter * QCOL, QCOL),
                       :].astype(jnp.bfloat16),
                preferred_element_type=jnp.float32,
            )
            if quarter == 0:
                out_ref[...] = contrib
            else:
                out_ref[...] += contrib
            if quarter >= 1:
                drain(quarter - 1)
        drain(3)

        for q in range(4):
            for s in sends[q]:
                s.wait_send()

    out2d = pl.pallas_call(
        body,
        out_shape=jax.ShapeDtypeStruct((SQ, D_MODEL), jnp.float32),
        in_specs=[
            pl.BlockSpec(memory_space=pltpu.VMEM),
            pl.BlockSpec(memory_space=pl.ANY),
            pl.BlockSpec(memory_space=pl.ANY),
            pl.BlockSpec(memory_space=pl.ANY),
            pl.BlockSpec(memory_space=pl.ANY),
        ],
        out_specs=pl.BlockSpec(memory_space=pltpu.VMEM),
        scratch_shapes=[
            pltpu.VMEM((D_MODEL, H_LOC * DH), jnp.float32),
            pltpu.VMEM((4, 2, NT, QB, DH), jnp.float32),
            pltpu.VMEM((4, 2, NT, QB, DH), jnp.float32),
            pltpu.VMEM((SKV, D_MODEL), jnp.float32),
            pltpu.SemaphoreType.DMA,
            pltpu.SemaphoreType.DMA,
            pltpu.SemaphoreType.DMA((4,)),
            pltpu.SemaphoreType.DMA((4,)),
            pltpu.VMEM((4, SQ, QCOL), jnp.bfloat16),
            pltpu.VMEM((4, 3, SQ, QCOL), jnp.bfloat16),
            pltpu.SemaphoreType.DMA((4, 3)),
            pltpu.SemaphoreType.DMA((4, 3)),
        ],
        compiler_params=pltpu.CompilerParams(
            collective_id=0, vmem_limit_bytes=60 * 1024 * 1024),
    )(x2d, Wq, k5, v5, Wo)

    return out2d.reshape(1, SQ, D_MODEL)
